# Initial kernel scaffold; baseline (speedup 1.0000x reference)
import jax
import jax.numpy as jnp
from jax.experimental import pallas as pl


def kernel(x_feat, W1, b1, g1, be1, W2, b2, Wp1, bp1, gp, bep, Wp2, bp2, clustering_labels, edge_index, batch):
    raise NotImplementedError("write your pallas kernel here")


# speedup vs baseline: 3.3577x; 3.3577x over previous
import functools

import jax
import jax.numpy as jnp
from jax import lax
from jax.experimental import pallas as pl
from jax.experimental.pallas import tpu as pltpu
from jax.experimental.pallas import tpu_sc as plsc

N = 10000
E = 160000
D = 256
L = 3
C = 4
G = 64
OUT = 256

NSC = 2
NTILE = 16
NW = NSC * NTILE
LANES = 16
RPT = 320
K = 128
E_PAD = E + K
LAST_ROWS = N - (NW - 1) * RPT
BR = 1000
NB = N // BR
BN_EPS = 1e-5


def _sc_body(val_hbm, agg_in_hbm, srcs_hbm, dsts_hbm, bounds_hbm, agg_out_hbm,
             src_v, dst_v, rel_v, bnd_v, rows_v, slab, sem):
  u = lax.axis_index("c") * NTILE + lax.axis_index("s")
  row_base = u * RPT

  @pl.when(u < NW - 1)
  def _():
    pltpu.sync_copy(agg_in_hbm.at[pl.ds(row_base, RPT)],
                    slab.at[pl.ds(0, RPT)])

  @pl.when(u == NW - 1)
  def _():
    pltpu.sync_copy(agg_in_hbm.at[pl.ds(row_base, LAST_ROWS)],
                    slab.at[pl.ds(0, LAST_ROWS)])

  pltpu.sync_copy(bounds_hbm.at[u], bnd_v)
  lo = bnd_v[0][0]
  hi = bnd_v[1][0]
  lo8 = jnp.bitwise_and(lo, -8)
  ntrips = jnp.maximum(0, (hi - lo8 + K - 1) // K)

  def trip(i, carry):
    base = pl.multiple_of(lo8 + i * K, 8)
    pltpu.sync_copy(srcs_hbm.at[pl.ds(base, K)], src_v)
    pltpu.sync_copy(dsts_hbm.at[pl.ds(base, K)], dst_v)
    for j in range(K // LANES):
      d = dst_v[pl.ds(j * LANES, LANES)]
      eidx = base + j * LANES + lax.broadcasted_iota(jnp.int32, (LANES,), 0)
      valid = (eidx >= lo) & (eidx < hi)
      rel_v[pl.ds(j * LANES, LANES)] = jnp.where(valid, d - row_base, RPT)
    pltpu.async_copy(val_hbm.at[src_v], rows_v, sem).wait()

    def group(g, c):
      relv = rel_v[pl.ds(g * LANES, LANES)]
      for e in range(LANES):
        rel = relv[e]
        ge = g * LANES + e
        for jj in range(D // LANES):
          sl = pl.ds(jj * LANES, LANES)
          plsc.addupdate(slab.at[rel, sl], rows_v[ge, sl])
      return c

    lax.fori_loop(0, K // LANES, group, 0)
    return carry

  lax.fori_loop(0, ntrips, trip, 0)

  @pl.when(u < NW - 1)
  def _():
    pltpu.sync_copy(slab.at[pl.ds(0, RPT)],
                    agg_out_hbm.at[pl.ds(row_base, RPT)])

  @pl.when(u == NW - 1)
  def _():
    pltpu.sync_copy(slab.at[pl.ds(0, LAST_ROWS)],
                    agg_out_hbm.at[pl.ds(row_base, LAST_ROWS)])


@functools.cache
def _get_sc_scatter():
  return pl.kernel(
    _sc_body,
    out_type=jax.ShapeDtypeStruct((N, D), jnp.float32),
    mesh=plsc.VectorSubcoreMesh(core_axis_name="c", subcore_axis_name="s",
                                num_cores=NSC, num_subcores=NTILE),
    scratch_types=[
        pltpu.VMEM((K,), jnp.int32),
        pltpu.VMEM((K,), jnp.int32),
        pltpu.VMEM((K,), jnp.int32),
        pltpu.VMEM((2, LANES), jnp.int32),
        pltpu.VMEM((K, D), jnp.float32),
        pltpu.VMEM((RPT + 8, D), jnp.float32),
        pltpu.SemaphoreType.DMA,
    ],
  )


def _pass_a_body(agg_ref, x_ref, w1_ref, b1_ref, h1_ref, sums_ref):
  i = pl.program_id(0)
  out = agg_ref[...] + x_ref[...]
  h1 = jnp.dot(out, w1_ref[...], preferred_element_type=jnp.float32)
  h1 = h1 + b1_ref[...]
  h1_ref[...] = h1
  part = jnp.concatenate([jnp.sum(h1, axis=0, keepdims=True),
                          jnp.sum(h1 * h1, axis=0, keepdims=True)], axis=0)

  @pl.when(i == 0)
  def _():
    sums_ref[...] = part

  @pl.when(i > 0)
  def _():
    sums_ref[...] += part


_pass_a = pl.pallas_call(
    _pass_a_body,
    grid=(NB,),
    in_specs=[
        pl.BlockSpec((BR, D), lambda i: (i, 0)),
        pl.BlockSpec((BR, D), lambda i: (i, 0)),
        pl.BlockSpec((D, D), lambda i: (0, 0)),
        pl.BlockSpec((1, D), lambda i: (0, 0)),
    ],
    out_specs=[
        pl.BlockSpec((BR, D), lambda i: (i, 0)),
        pl.BlockSpec((2, D), lambda i: (0, 0)),
    ],
    out_shape=[
        jax.ShapeDtypeStruct((N, D), jnp.float32),
        jax.ShapeDtypeStruct((2, D), jnp.float32),
    ],
)


def _bn_relu_w2(h1_ref, x_ref, m_ref, sums_ref, g_ref, be_ref, w2_ref, b2_ref):
  mu = sums_ref[0:1, :] / N
  var = sums_ref[1:2, :] / N - mu * mu
  rstd = lax.rsqrt(var + BN_EPS)
  hh = jnp.maximum(g_ref[...] * (h1_ref[...] - mu) * rstd + be_ref[...], 0.0)
  h2 = jnp.dot(hh, w2_ref[...], preferred_element_type=jnp.float32)
  h2 = h2 + b2_ref[...]
  m = m_ref[...] > 0.0
  xn = jnp.where(m, h2, x_ref[...])
  dl = jnp.where(m, h2 - x_ref[...], 0.0)
  return xn, dl


def _pass_b_pool_body(h1_ref, x_ref, m_ref, sums_ref, g_ref, be_ref, w2_ref,
                      b2_ref, xn_ref, dl_ref, pool_ref):
  i = pl.program_id(0)
  xn, dl = _bn_relu_w2(h1_ref, x_ref, m_ref, sums_ref, g_ref, be_ref,
                       w2_ref, b2_ref)
  xn_ref[...] = xn
  dl_ref[...] = dl
  gids = lax.broadcasted_iota(jnp.int32, (G, BR), 0)
  rows = lax.broadcasted_iota(jnp.int32, (G, BR), 1) + i * BR
  seg = (rows * G) // N
  s_mat = (seg == gids).astype(jnp.float32)
  contrib = jnp.dot(s_mat, xn, preferred_element_type=jnp.float32)

  @pl.when(i == 0)
  def _():
    pool_ref[...] = contrib

  @pl.when(i > 0)
  def _():
    pool_ref[...] += contrib


_B_IN_SPECS = [
    pl.BlockSpec((BR, D), lambda i: (i, 0)),
    pl.BlockSpec((BR, D), lambda i: (i, 0)),
    pl.BlockSpec((BR, 1), lambda i: (i, 0)),
    pl.BlockSpec((2, D), lambda i: (0, 0)),
    pl.BlockSpec((1, D), lambda i: (0, 0)),
    pl.BlockSpec((1, D), lambda i: (0, 0)),
    pl.BlockSpec((D, D), lambda i: (0, 0)),
    pl.BlockSpec((1, D), lambda i: (0, 0)),
]

_pass_b_pool = pl.pallas_call(
    _pass_b_pool_body,
    grid=(NB,),
    in_specs=_B_IN_SPECS,
    out_specs=[
        pl.BlockSpec((BR, D), lambda i: (i, 0)),
        pl.BlockSpec((BR, D), lambda i: (i, 0)),
        pl.BlockSpec((G, D), lambda i: (0, 0)),
    ],
    out_shape=[
        jax.ShapeDtypeStruct((N, D), jnp.float32),
        jax.ShapeDtypeStruct((N, D), jnp.float32),
        jax.ShapeDtypeStruct((G, D), jnp.float32),
    ],
)


def _head_body(xg_ref, wp1_ref, bp1_ref, gp_ref, bep_ref, wp2_ref, bp2_ref,
               out_ref):
  h = jnp.dot(xg_ref[...], wp1_ref[...], preferred_element_type=jnp.float32)
  h = h + bp1_ref[...]
  mu = jnp.mean(h, axis=0, keepdims=True)
  var = jnp.mean((h - mu) * (h - mu), axis=0, keepdims=True)
  h = gp_ref[...] * (h - mu) * lax.rsqrt(var + BN_EPS) + bep_ref[...]
  h = jnp.maximum(h, 0.0)
  o = jnp.dot(h, wp2_ref[...], preferred_element_type=jnp.float32)
  out_ref[...] = o + bp2_ref[...]


_head = pl.pallas_call(
    _head_body,
    out_shape=jax.ShapeDtypeStruct((G, OUT), jnp.float32),
)


def kernel(x_feat, W1, b1, g1, be1, W2, b2, Wp1, bp1, gp, bep, Wp2, bp2,
           clustering_labels, edge_index, batch):
  lab = clustering_labels.astype(jnp.int32)
  src = edge_index[0].astype(jnp.int32)
  dst = edge_index[1].astype(jnp.int32)

  key = (dst // RPT) * C + lab[src]
  order = jnp.argsort(key)
  key_s = key[order]
  src_s = jnp.concatenate([src[order], jnp.zeros((E_PAD - E,), jnp.int32)])
  dst_s = jnp.concatenate([dst[order], jnp.zeros((E_PAD - E,), jnp.int32)])
  offs = jnp.searchsorted(key_s, jnp.arange(NW * C + 1, dtype=jnp.int32)
                          ).astype(jnp.int32)

  tiles = jnp.arange(NW, dtype=jnp.int32)

  def make_bounds(lo, hi):
    return jnp.broadcast_to(jnp.stack([lo, hi], axis=1)[:, :, None],
                            (NW, 2, LANES)).astype(jnp.int32)

  bounds_full = make_bounds(offs[tiles * C], offs[tiles * C + C])
  bounds_c = [make_bounds(offs[tiles * C + c], offs[tiles * C + c + 1])
              for c in range(C)]

  masks = (lab[:, None] == jnp.arange(C, dtype=jnp.int32)[None, :]
           ).astype(jnp.float32)

  sc_scatter = _get_sc_scatter()
  x = x_feat.astype(jnp.float32)
  zeros_nd = jnp.zeros((N, D), jnp.float32)
  agg = sc_scatter(x, zeros_nd, src_s, dst_s, bounds_full)

  b1r = b1.reshape(L * C, 1, D)
  b2r = b2.reshape(L * C, 1, D)
  g1r = g1.reshape(L * C, 1, D)
  be1r = be1.reshape(L * C, 1, D)
  mask_stack = jnp.stack([masks[:, i % C:i % C + 1] for i in range(L * C)])
  bounds_stack = jnp.stack([bounds_c[i % C] for i in range(L * C)])

  def step(carry, xs):
    x, agg = carry
    w1_i, b1_i, g1_i, be1_i, w2_i, b2_i, m_i, bnd_i = xs
    h1, sums = _pass_a(agg, x, w1_i, b1_i)
    x, delta, pool_i = _pass_b_pool(h1, x, m_i, sums, g1_i, be1_i, w2_i, b2_i)
    agg = sc_scatter(delta, agg, src_s, dst_s, bnd_i)
    return (x, agg), pool_i

  (_, _), pools = lax.scan(
      step, (x, agg),
      (W1, b1r, g1r, be1r, W2, b2r, mask_stack, bounds_stack))

  xg = jnp.concatenate([pools[C - 1], pools[2 * C - 1], pools[3 * C - 1]],
                       axis=1)
  return _head(xg, Wp1, bp1.reshape(1, D), gp.reshape(1, D),
               bep.reshape(1, D), Wp2, bp2.reshape(1, OUT))


# speedup vs baseline: 3.4413x; 1.0249x over previous
import functools

import jax
import jax.numpy as jnp
from jax import lax
from jax.experimental import pallas as pl
from jax.experimental.pallas import tpu as pltpu
from jax.experimental.pallas import tpu_sc as plsc

N = 10000
E = 160000
D = 256
L = 3
C = 4
G = 64
OUT = 256

NSC = 2
NTILE = 16
NW = NSC * NTILE
LANES = 16
RPT = 320
K = 128
E_PAD = E + K
LAST_ROWS = N - (NW - 1) * RPT
BR = 1000
NB = N // BR
BN_EPS = 1e-5


def _sc_body(val_hbm, agg_in_hbm, srcs_hbm, dsts_hbm, bounds_hbm, agg_out_hbm,
             src_v, dst_v, rel_v, bnd_v, rows_v, slab, sem):
  u = lax.axis_index("c") * NTILE + lax.axis_index("s")
  row_base = u * RPT

  @pl.when(u < NW - 1)
  def _():
    pltpu.sync_copy(agg_in_hbm.at[pl.ds(row_base, RPT)],
                    slab.at[pl.ds(0, RPT)])

  @pl.when(u == NW - 1)
  def _():
    pltpu.sync_copy(agg_in_hbm.at[pl.ds(row_base, LAST_ROWS)],
                    slab.at[pl.ds(0, LAST_ROWS)])

  pltpu.sync_copy(bounds_hbm.at[u], bnd_v)
  lo = bnd_v[0][0]
  hi = bnd_v[1][0]
  lo8 = jnp.bitwise_and(lo, -8)
  ntrips = jnp.maximum(0, (hi - lo8 + K - 1) // K)

  def trip(i, carry):
    base = pl.multiple_of(lo8 + i * K, 8)
    pltpu.sync_copy(srcs_hbm.at[pl.ds(base, K)], src_v)
    pltpu.sync_copy(dsts_hbm.at[pl.ds(base, K)], dst_v)
    for j in range(K // LANES):
      d = dst_v[pl.ds(j * LANES, LANES)]
      eidx = base + j * LANES + lax.broadcasted_iota(jnp.int32, (LANES,), 0)
      valid = (eidx >= lo) & (eidx < hi)
      rel_v[pl.ds(j * LANES, LANES)] = jnp.where(valid, d - row_base, RPT)
    pltpu.async_copy(val_hbm.at[src_v], rows_v, sem).wait()

    def group(g, c):
      relv = rel_v[pl.ds(g * LANES, LANES)]
      for e in range(LANES):
        rel = relv[e]
        ge = g * LANES + e
        for jj in range(D // LANES):
          sl = pl.ds(jj * LANES, LANES)
          plsc.addupdate(slab.at[rel, sl], rows_v[ge, sl])
      return c

    lax.fori_loop(0, K // LANES, group, 0)
    return carry

  lax.fori_loop(0, ntrips, trip, 0)

  @pl.when(u < NW - 1)
  def _():
    pltpu.sync_copy(slab.at[pl.ds(0, RPT)],
                    agg_out_hbm.at[pl.ds(row_base, RPT)])

  @pl.when(u == NW - 1)
  def _():
    pltpu.sync_copy(slab.at[pl.ds(0, LAST_ROWS)],
                    agg_out_hbm.at[pl.ds(row_base, LAST_ROWS)])


@functools.cache
def _get_sc_scatter():
  return pl.kernel(
    _sc_body,
    out_type=jax.ShapeDtypeStruct((N, D), jnp.float32),
    mesh=plsc.VectorSubcoreMesh(core_axis_name="c", subcore_axis_name="s",
                                num_cores=NSC, num_subcores=NTILE),
    scratch_types=[
        pltpu.VMEM((K,), jnp.int32),
        pltpu.VMEM((K,), jnp.int32),
        pltpu.VMEM((K,), jnp.int32),
        pltpu.VMEM((2, LANES), jnp.int32),
        pltpu.VMEM((K, D), jnp.float32),
        pltpu.VMEM((RPT + 8, D), jnp.float32),
        pltpu.SemaphoreType.DMA,
    ],
  )


def _pass_a_body(agg_ref, x_ref, w1_ref, b1_ref, h1_ref, sums_ref):
  i = pl.program_id(0)
  out = agg_ref[...] + x_ref[...]
  h1 = jnp.dot(out, w1_ref[...], preferred_element_type=jnp.float32)
  h1 = h1 + b1_ref[...]
  h1_ref[...] = h1
  part = jnp.concatenate([jnp.sum(h1, axis=0, keepdims=True),
                          jnp.sum(h1 * h1, axis=0, keepdims=True)], axis=0)

  @pl.when(i == 0)
  def _():
    sums_ref[...] = part

  @pl.when(i > 0)
  def _():
    sums_ref[...] += part


_pass_a = pl.pallas_call(
    _pass_a_body,
    grid=(NB,),
    in_specs=[
        pl.BlockSpec((BR, D), lambda i: (i, 0)),
        pl.BlockSpec((BR, D), lambda i: (i, 0)),
        pl.BlockSpec((D, D), lambda i: (0, 0)),
        pl.BlockSpec((1, D), lambda i: (0, 0)),
    ],
    out_specs=[
        pl.BlockSpec((BR, D), lambda i: (i, 0)),
        pl.BlockSpec((2, D), lambda i: (0, 0)),
    ],
    out_shape=[
        jax.ShapeDtypeStruct((N, D), jnp.float32),
        jax.ShapeDtypeStruct((2, D), jnp.float32),
    ],
)


def _bn_relu_w2(h1_ref, x_ref, m_ref, sums_ref, g_ref, be_ref, w2_ref, b2_ref):
  mu = sums_ref[0:1, :] / N
  var = sums_ref[1:2, :] / N - mu * mu
  rstd = lax.rsqrt(var + BN_EPS)
  hh = jnp.maximum(g_ref[...] * (h1_ref[...] - mu) * rstd + be_ref[...], 0.0)
  h2 = jnp.dot(hh, w2_ref[...], preferred_element_type=jnp.float32)
  h2 = h2 + b2_ref[...]
  m = m_ref[...] > 0.0
  xn = jnp.where(m, h2, x_ref[...])
  dl = jnp.where(m, h2 - x_ref[...], 0.0)
  return xn, dl


def _pass_b_pool_body(h1_ref, x_ref, m_ref, sums_ref, g_ref, be_ref, w2_ref,
                      b2_ref, xn_ref, dl_ref, pool_ref):
  i = pl.program_id(0)
  xn, dl = _bn_relu_w2(h1_ref, x_ref, m_ref, sums_ref, g_ref, be_ref,
                       w2_ref, b2_ref)
  xn_ref[...] = xn
  dl_ref[...] = dl
  gids = lax.broadcasted_iota(jnp.int32, (G, BR), 0)
  rows = lax.broadcasted_iota(jnp.int32, (G, BR), 1) + i * BR
  seg = (rows * G) // N
  s_mat = (seg == gids).astype(jnp.float32)
  contrib = jnp.dot(s_mat, xn, preferred_element_type=jnp.float32)

  @pl.when(i == 0)
  def _():
    pool_ref[...] = contrib

  @pl.when(i > 0)
  def _():
    pool_ref[...] += contrib


_B_IN_SPECS = [
    pl.BlockSpec((BR, D), lambda i: (i, 0)),
    pl.BlockSpec((BR, D), lambda i: (i, 0)),
    pl.BlockSpec((BR, 1), lambda i: (i, 0)),
    pl.BlockSpec((2, D), lambda i: (0, 0)),
    pl.BlockSpec((1, D), lambda i: (0, 0)),
    pl.BlockSpec((1, D), lambda i: (0, 0)),
    pl.BlockSpec((D, D), lambda i: (0, 0)),
    pl.BlockSpec((1, D), lambda i: (0, 0)),
]

_pass_b_pool = pl.pallas_call(
    _pass_b_pool_body,
    grid=(NB,),
    in_specs=_B_IN_SPECS,
    out_specs=[
        pl.BlockSpec((BR, D), lambda i: (i, 0)),
        pl.BlockSpec((BR, D), lambda i: (i, 0)),
        pl.BlockSpec((G, D), lambda i: (0, 0)),
    ],
    out_shape=[
        jax.ShapeDtypeStruct((N, D), jnp.float32),
        jax.ShapeDtypeStruct((N, D), jnp.float32),
        jax.ShapeDtypeStruct((G, D), jnp.float32),
    ],
)


def _head_body(xg_ref, wp1_ref, bp1_ref, gp_ref, bep_ref, wp2_ref, bp2_ref,
               out_ref):
  h = jnp.dot(xg_ref[...], wp1_ref[...], preferred_element_type=jnp.float32)
  h = h + bp1_ref[...]
  mu = jnp.mean(h, axis=0, keepdims=True)
  var = jnp.mean((h - mu) * (h - mu), axis=0, keepdims=True)
  h = gp_ref[...] * (h - mu) * lax.rsqrt(var + BN_EPS) + bep_ref[...]
  h = jnp.maximum(h, 0.0)
  o = jnp.dot(h, wp2_ref[...], preferred_element_type=jnp.float32)
  out_ref[...] = o + bp2_ref[...]


_head = pl.pallas_call(
    _head_body,
    out_shape=jax.ShapeDtypeStruct((G, OUT), jnp.float32),
)


def kernel(x_feat, W1, b1, g1, be1, W2, b2, Wp1, bp1, gp, bep, Wp2, bp2,
           clustering_labels, edge_index, batch):
  lab = clustering_labels.astype(jnp.int32)
  src = edge_index[0].astype(jnp.int32)
  dst = edge_index[1].astype(jnp.int32)

  key = (dst // RPT) * C + lab[src]
  packed = (key << 18) | jnp.arange(E, dtype=jnp.int32)
  sp = jnp.sort(packed)
  order = sp & ((1 << 18) - 1)
  key_s = sp >> 18
  src_s = jnp.concatenate([src[order], jnp.zeros((E_PAD - E,), jnp.int32)])
  dst_s = jnp.concatenate([dst[order], jnp.zeros((E_PAD - E,), jnp.int32)])
  offs = jnp.searchsorted(key_s, jnp.arange(NW * C + 1, dtype=jnp.int32)
                          ).astype(jnp.int32)

  tiles = jnp.arange(NW, dtype=jnp.int32)

  def make_bounds(lo, hi):
    return jnp.broadcast_to(jnp.stack([lo, hi], axis=1)[:, :, None],
                            (NW, 2, LANES)).astype(jnp.int32)

  bounds_full = make_bounds(offs[tiles * C], offs[tiles * C + C])
  bounds_c = [make_bounds(offs[tiles * C + c], offs[tiles * C + c + 1])
              for c in range(C)]

  masks = (lab[:, None] == jnp.arange(C, dtype=jnp.int32)[None, :]
           ).astype(jnp.float32)

  sc_scatter = _get_sc_scatter()
  x = x_feat.astype(jnp.float32)
  zeros_nd = jnp.zeros((N, D), jnp.float32)
  agg = sc_scatter(x, zeros_nd, src_s, dst_s, bounds_full)

  b1r = b1.reshape(L * C, 1, D)
  b2r = b2.reshape(L * C, 1, D)
  g1r = g1.reshape(L * C, 1, D)
  be1r = be1.reshape(L * C, 1, D)
  mask_stack = jnp.stack([masks[:, i % C:i % C + 1] for i in range(L * C)])
  bounds_stack = jnp.stack([bounds_c[i % C] for i in range(L * C)])

  def step(carry, xs):
    x, agg = carry
    w1_i, b1_i, g1_i, be1_i, w2_i, b2_i, m_i, bnd_i = xs
    h1, sums = _pass_a(agg, x, w1_i, b1_i)
    x, delta, pool_i = _pass_b_pool(h1, x, m_i, sums, g1_i, be1_i, w2_i, b2_i)
    agg = sc_scatter(delta, agg, src_s, dst_s, bnd_i)
    return (x, agg), pool_i

  (_, _), pools = lax.scan(
      step, (x, agg),
      (W1, b1r, g1r, be1r, W2, b2r, mask_stack, bounds_stack))

  xg = jnp.concatenate([pools[C - 1], pools[2 * C - 1], pools[3 * C - 1]],
                       axis=1)
  return _head(xg, Wp1, bp1.reshape(1, D), gp.reshape(1, D),
               bep.reshape(1, D), Wp2, bp2.reshape(1, OUT))
